# baseline (speedup 1.0000x reference)
import functools

import jax
import jax.numpy as jnp
from jax import lax
from jax.experimental import pallas as pl
from jax.experimental.pallas import tpu as pltpu
from jax.experimental.pallas import tpu_sc as plsc

_NUM_CLASSES = 1000000
_HIDDEN = 64
_BATCH = 16384
_DROPOUT_PROB = 0.1

_NC = 2
_NS = 16
_NW = _NC * _NS
_B_PER_W = _BATCH // _NW
_IDX_ROW = 128
_ROWS_PER_W = _B_PER_W // _IDX_ROW


def _make_gather():
    mesh = plsc.VectorSubcoreMesh(
        core_axis_name="c", subcore_axis_name="s",
        num_cores=_NC, num_subcores=_NS)

    @functools.partial(
        pl.kernel,
        out_type=jax.ShapeDtypeStruct((_BATCH, _HIDDEN), jnp.float32),
        mesh=mesh,
        scratch_types=[
            pltpu.VMEM((_ROWS_PER_W, _IDX_ROW), jnp.int32),
            pltpu.VMEM((_B_PER_W, _HIDDEN), jnp.float32),
            pltpu.SemaphoreType.DMA,
        ],
        compiler_params=pltpu.CompilerParams(use_tc_tiling_on_sc=False),
    )
    def gather_kernel(lab_hbm, table_hbm, out_hbm, idx_v, rows_v, sem):
        wid = lax.axis_index("s") * _NC + lax.axis_index("c")
        pltpu.sync_copy(lab_hbm.at[pl.ds(wid * _ROWS_PER_W, _ROWS_PER_W)],
                        idx_v)
        copies = []
        for j in range(_ROWS_PER_W):
            copies.append(pltpu.async_copy(
                table_hbm.at[idx_v.at[j]],
                rows_v.at[pl.ds(j * _IDX_ROW, _IDX_ROW)],
                sem))
        for c in copies:
            c.wait()
        pltpu.sync_copy(rows_v, out_hbm.at[pl.ds(wid * _B_PER_W, _B_PER_W)])

    return gather_kernel


_gather = _make_gather()


def kernel(labels, train, embedding_table):
    drop_key = jax.random.key(1)
    drop_ids = jax.random.uniform(drop_key, (labels.shape[0],)) < _DROPOUT_PROB
    do_drop = jnp.asarray(train) != 0
    lab = jnp.where(do_drop & drop_ids, _NUM_CLASSES, labels)
    lab2d = lab.reshape(_NW * _ROWS_PER_W, _IDX_ROW)
    return _gather(lab2d, embedding_table)


# speedup vs baseline: 1.6176x; 1.6176x over previous
import functools

import jax
import jax.numpy as jnp
from jax import lax
from jax.experimental import pallas as pl
from jax.experimental.pallas import tpu as pltpu
from jax.experimental.pallas import tpu_sc as plsc

_NUM_CLASSES = 1000000
_HIDDEN = 64
_BATCH = 16384
_DROPOUT_PROB = 0.1

_NC = 2
_NS = 16
_NW = _NC * _NS
_B_PER_W = _BATCH // _NW
_K = 16


def _make_gather():
    mesh = plsc.VectorSubcoreMesh(
        core_axis_name="c", subcore_axis_name="s",
        num_cores=_NC, num_subcores=_NS)

    @functools.partial(
        pl.kernel,
        out_type=jax.ShapeDtypeStruct((_BATCH, _HIDDEN), jnp.float32),
        mesh=mesh,
        scratch_types=[
            pltpu.VMEM((_B_PER_W,), jnp.int32),
            pltpu.VMEM((_B_PER_W, _HIDDEN), jnp.float32),
            pltpu.SemaphoreType.DMA,
        ],
    )
    def gather_kernel(lab_hbm, table_hbm, out_hbm, idx_v, rows_v, sem):
        wid = lax.axis_index("s") * _NC + lax.axis_index("c")
        base = wid * _B_PER_W
        pltpu.sync_copy(lab_hbm.at[pl.ds(base, _B_PER_W)], idx_v)

        @pl.loop(0, _B_PER_W, step=_K)
        def _(g):
            v = idx_v[pl.ds(g, _K)]
            copies = []
            for j in range(_K):
                copies.append(pltpu.async_copy(
                    table_hbm.at[pl.ds(v[j], 1)],
                    rows_v.at[pl.ds(g + j, 1)],
                    sem))
            for c in copies:
                c.wait()
        pltpu.sync_copy(rows_v, out_hbm.at[pl.ds(base, _B_PER_W)])

    return gather_kernel


_gather = _make_gather()


def kernel(labels, train, embedding_table):
    drop_key = jax.random.key(1)
    drop_ids = jax.random.uniform(drop_key, (labels.shape[0],)) < _DROPOUT_PROB
    do_drop = jnp.asarray(train) != 0
    lab = jnp.where(do_drop & drop_ids, _NUM_CLASSES, labels)
    return _gather(lab, embedding_table)
